# baseline (speedup 1.0000x reference)
import functools

import jax
import jax.numpy as jnp
from jax.experimental import pallas as pl
from jax.experimental.pallas import tpu as pltpu


def _decode_body(x_ref, a_ref, o_ref, *, nG, nA, nCH, bb):
    nGG = x_ref.shape[2]
    col = jax.lax.broadcasted_iota(jnp.int32, (1, nGG), 1)
    scale = a_ref[0, 0, 2]
    xc = (col % nG).astype(jnp.float32) * scale
    yc = (col // nG).astype(jnp.float32) * scale
    xy_off = jnp.concatenate([xc, yc], axis=0)
    for b in range(bb):
        v = x_ref[b]
        sig = jax.nn.sigmoid(v)
        pieces = []
        for a in range(nA):
            base = a * nCH
            xy = sig[base:base + 2, :] * scale + xy_off
            e = jnp.exp(v[base + 2:base + 4, :])
            wh = jnp.concatenate(
                [e[0:1, :] * a_ref[a, 0, 0], e[1:2, :] * a_ref[a, 0, 1]],
                axis=0)
            pieces += [xy, wh, sig[base + 4:base + nCH, :]]
        out = jnp.concatenate(pieces, axis=0)
        ng = nGG // 16
        for a in range(nA):
            t3 = jnp.transpose(
                out[a * nCH:(a + 1) * nCH, :].reshape(nCH, ng, 16), (1, 2, 0))
            o_ref[b, pl.ds(a * ng, ng), :] = t3.reshape(ng, 16 * nCH)


def kernel(raw, anchors, img_size):
    nB, C, nG, _ = raw.shape
    nA = anchors.shape[0]
    nCH = C // nA
    nGG = nG * nG
    scale = (jnp.float32(img_size) / jnp.float32(nG)).reshape(1, 1)

    x = raw.reshape(nB, C, nGG)
    anch = jnp.concatenate(
        [anchors, jnp.broadcast_to(scale, (nA, 1)),
         jnp.zeros((nA, 1), jnp.float32)], axis=1).reshape(nA, 1, 4)

    bb = 1
    body = functools.partial(_decode_body, nG=nG, nA=nA, nCH=nCH, bb=bb)

    out = pl.pallas_call(
        body,
        grid=(nB // bb,),
        in_specs=[
            pl.BlockSpec((bb, C, nGG), lambda b: (b, 0, 0)),
            pl.BlockSpec((nA, 1, 4), lambda b: (0, 0, 0)),
        ],
        out_specs=pl.BlockSpec((bb, nA * nGG // 16, 16 * nCH),
                               lambda b: (b, 0, 0)),
        out_shape=jax.ShapeDtypeStruct((nB, nA * nGG // 16, 16 * nCH),
                                       jnp.float32),
        compiler_params=pltpu.CompilerParams(
            dimension_semantics=("parallel",),
            vmem_limit_bytes=100 * 1024 * 1024,
        ),
    )(x, anch)
    return out.reshape(nB, nA * nGG, nCH)


# speedup vs baseline: 3.1625x; 3.1625x over previous
import functools

import jax
import jax.numpy as jnp
from jax.experimental import pallas as pl
from jax.experimental.pallas import tpu as pltpu


def _decode_body(x_ref, a_ref, o_ref, *, nG, nA, nCH, bb):
    nGG = x_ref.shape[2]
    col = jax.lax.broadcasted_iota(jnp.int32, (1, nGG), 1)
    scale = a_ref[0, 0, 2]
    xc = (col % nG).astype(jnp.float32) * scale
    yc = (col // nG).astype(jnp.float32) * scale
    xy_off = jnp.concatenate([xc, yc], axis=0)
    for b in range(bb):
        v = x_ref[b]
        sig = jax.nn.sigmoid(v)
        pieces = []
        for a in range(nA):
            base = a * nCH
            xy = sig[base:base + 2, :] * scale + xy_off
            e = jnp.exp(v[base + 2:base + 4, :])
            wh = jnp.concatenate(
                [e[0:1, :] * a_ref[a, 0, 0], e[1:2, :] * a_ref[a, 0, 1]],
                axis=0)
            pieces += [xy, wh, sig[base + 4:base + nCH, :]]
        out = jnp.concatenate(pieces, axis=0)
        for a in range(nA):
            o_ref[b, pl.ds(a * nGG, nGG), :] = out[a * nCH:(a + 1) * nCH, :].T


def kernel(raw, anchors, img_size):
    nB, C, nG, _ = raw.shape
    nA = anchors.shape[0]
    nCH = C // nA
    nGG = nG * nG
    scale = (jnp.float32(img_size) / jnp.float32(nG)).reshape(1, 1)

    x = raw.reshape(nB, C, nGG)
    anch = jnp.concatenate(
        [anchors, jnp.broadcast_to(scale, (nA, 1)),
         jnp.zeros((nA, 1), jnp.float32)], axis=1).reshape(nA, 1, 4)

    bb = 2
    body = functools.partial(_decode_body, nG=nG, nA=nA, nCH=nCH, bb=bb)

    out = pl.pallas_call(
        body,
        grid=(nB // bb,),
        in_specs=[
            pl.BlockSpec((bb, C, nGG), lambda b: (b, 0, 0)),
            pl.BlockSpec((nA, 1, 4), lambda b: (0, 0, 0)),
        ],
        out_specs=pl.BlockSpec((bb, nA * nGG, nCH), lambda b: (b, 0, 0)),
        out_shape=jax.ShapeDtypeStruct((nB, nA * nGG, nCH), jnp.float32),
        compiler_params=pltpu.CompilerParams(
            dimension_semantics=("parallel",),
            vmem_limit_bytes=100 * 1024 * 1024,
        ),
    )(x, anch)
    return out
